# baseline (speedup 1.0000x reference)
import functools

import jax
import jax.numpy as jnp
from jax.experimental import pallas as pl
from jax.experimental.pallas import tpu as pltpu

_B = 16
_C = 64
_HW = 64 * 64
_K = 1024
_GR = _C + 8


def _vq_pipeline(z_hbm, emb_ref, gmat_ref, quant_hbm, idx_hbm, loss_ref,
                 zbuf, qbuf, ibuf, in_sem, outq_sem, outi_sem):
    emb = emb_ref[...]
    gmat = gmat_ref[...]
    h = 0.5 * jnp.sum(emb * emb, axis=0)

    def in_copy(i):
        return pltpu.make_async_copy(
            z_hbm.at[i], zbuf.at[i % 2], in_sem.at[i % 2])

    def outq_copy(i):
        return pltpu.make_async_copy(
            qbuf.at[i % 2], quant_hbm.at[i], outq_sem.at[i % 2])

    def outi_copy(i):
        return pltpu.make_async_copy(
            ibuf.at[i % 2], idx_hbm.at[i], outi_sem.at[i % 2])

    in_copy(0).start()
    loss_acc = jnp.zeros((_HW,), jnp.float32)
    for i in range(_B):
        if i + 1 < _B:
            in_copy(i + 1).start()
        in_copy(i).wait()
        zb = zbuf[i % 2]
        scores = jax.lax.dot_general(
            emb, zb, (((0,), (0,)), ((), ())),
            preferred_element_type=jnp.float32)
        score = scores - h[:, None]
        m = jnp.max(score, axis=0)
        onehot = (score >= m[None, :]).astype(
            jnp.float32).astype(jnp.bfloat16)
        qa = jax.lax.dot_general(
            gmat, onehot, (((1,), (0,)), ((), ())),
            preferred_element_type=jnp.float32)
        quant = qa[:_C]
        idx = (qa[_C] * 32.0 + qa[_C + 1] + 0.5).astype(jnp.int32)
        if i >= 2:
            outq_copy(i - 2).wait()
            outi_copy(i - 2).wait()
        qbuf[i % 2] = quant
        ibuf[i % 2, 0] = idx
        outq_copy(i).start()
        outi_copy(i).start()
        diff = quant - zb
        loss_acc = loss_acc + jnp.sum(diff * diff, axis=0)
    outq_copy(_B - 2).wait()
    outi_copy(_B - 2).wait()
    outq_copy(_B - 1).wait()
    outi_copy(_B - 1).wait()
    loss_ref[0] = loss_acc


@jax.jit
def kernel(z, embedding):
    commitment_cost = 0.25
    z3 = z.reshape(_B, _C, _HW)
    ks = jax.lax.broadcasted_iota(jnp.int32, (1, _K), 1)
    gmat = jnp.concatenate(
        [embedding.astype(jnp.bfloat16),
         (ks >> 5).astype(jnp.bfloat16),
         (ks & 31).astype(jnp.bfloat16),
         jnp.zeros((_GR - _C - 2, _K), jnp.bfloat16)], axis=0)
    quant, idx, loss_parts = pl.pallas_call(
        _vq_pipeline,
        in_specs=[
            pl.BlockSpec(memory_space=pl.ANY),
            pl.BlockSpec(memory_space=pltpu.VMEM),
            pl.BlockSpec(memory_space=pltpu.VMEM),
        ],
        out_specs=[
            pl.BlockSpec(memory_space=pl.ANY),
            pl.BlockSpec(memory_space=pl.ANY),
            pl.BlockSpec(memory_space=pltpu.VMEM),
        ],
        out_shape=[
            jax.ShapeDtypeStruct((_B, _C, _HW), jnp.float32),
            jax.ShapeDtypeStruct((_B, 1, _HW), jnp.int32),
            jax.ShapeDtypeStruct((1, _HW), jnp.float32),
        ],
        scratch_shapes=[
            pltpu.VMEM((2, _C, _HW), jnp.float32),
            pltpu.VMEM((2, _C, _HW), jnp.float32),
            pltpu.VMEM((2, 1, _HW), jnp.int32),
            pltpu.SemaphoreType.DMA((2,)),
            pltpu.SemaphoreType.DMA((2,)),
            pltpu.SemaphoreType.DMA((2,)),
        ],
    )(z3, embedding, gmat)
    quantized_out = quant.reshape(z.shape)
    encoding_indices = idx.reshape(_B, 64, 64)
    loss = (1.0 + commitment_cost) * jnp.sum(loss_parts) / z.size
    return (quantized_out, loss, encoding_indices)
